# Initial kernel scaffold; baseline (speedup 1.0000x reference)
import functools

import jax
import jax.numpy as jnp
from jax import lax
from jax.experimental import pallas as pl
from jax.experimental.pallas import tpu as pltpu
from jax.experimental.pallas import tpu_sc as plsc

N = 10000
D = 128
C = 40
E = 320000

NW = 32
EPT = E // NW
CHUNK = 128
NCH = (EPT + CHUNK - 1) // CHUNK
EPT_PAD = NCH * CHUNK
N_PAD = 10240
RPT = N_PAD // 16
BLK = 512


def _proj_tail(h, as_ref, ad_ref, asrc_ref, adst_ref, g_ref, mx_ref):
    i = pl.program_id(0)
    asrc = jnp.sum(h * as_ref[...], axis=1, keepdims=True)
    adst = jnp.sum(h * ad_ref[...], axis=1, keepdims=True)
    asrc_ref[...] = asrc
    adst_ref[...] = adst

    @pl.when(i == 0)
    def _():
        mx_ref[0] = -jnp.inf
        mx_ref[1] = -jnp.inf

    mx_ref[0] = jnp.maximum(mx_ref[0], jnp.max(asrc))
    mx_ref[1] = jnp.maximum(mx_ref[1], jnp.max(adst))

    @pl.when(i == pl.num_programs(0) - 1)
    def _():
        s = mx_ref[0] + mx_ref[1]
        g = jnp.where(s >= 0, s, 0.2 * s)
        g_ref[...] = jnp.full((1, 16), g, jnp.float32)


def _tc1_body(x_ref, w1_ref, b1_ref, wg_ref, as_ref, ad_ref,
              h_ref, asrc_ref, adst_ref, g_ref, mx_ref):
    h0 = jnp.maximum(jnp.dot(x_ref[...], w1_ref[...],
                             preferred_element_type=jnp.float32) + b1_ref[...], 0.0)
    h = jnp.dot(h0, wg_ref[...], preferred_element_type=jnp.float32)
    h_ref[...] = h
    _proj_tail(h, as_ref, ad_ref, asrc_ref, adst_ref, g_ref, mx_ref)


def _tc2_body(acc_ref, den_ref, bg_ref, wg_ref, as_ref, ad_ref,
              h_ref, asrc_ref, adst_ref, g_ref, mx_ref):
    agg = acc_ref[0] + acc_ref[1]
    dn = den_ref[0] + den_ref[1]
    out = agg / (dn + 1e-16) + bg_ref[...]
    h1 = jnp.maximum(out, 0.0)
    h = jnp.dot(h1, wg_ref[...], preferred_element_type=jnp.float32)
    h_ref[...] = h
    _proj_tail(h, as_ref, ad_ref, asrc_ref, adst_ref, g_ref, mx_ref)


def _tc3_body(acc_ref, den_ref, bg_ref, w2_ref, b2_ref, o_ref):
    agg = acc_ref[0] + acc_ref[1]
    dn = den_ref[0] + den_ref[1]
    out = agg / (dn + 1e-16) + bg_ref[...]
    logits = jnp.dot(out, w2_ref[...],
                     preferred_element_type=jnp.float32) + b2_ref[...]
    m = jnp.max(logits, axis=1, keepdims=True)
    ls = logits - m
    o_ref[...] = ls - jnp.log(jnp.sum(jnp.exp(ls), axis=1, keepdims=True))


def _tc_proj1(x, w1, b1, wg, a_s, a_d):
    n = x.shape[0]
    return pl.pallas_call(
        _tc1_body,
        grid=(n // BLK,),
        in_specs=[
            pl.BlockSpec((BLK, D), lambda i: (i, 0)),
            pl.BlockSpec((D, D), lambda i: (0, 0)),
            pl.BlockSpec((1, D), lambda i: (0, 0)),
            pl.BlockSpec((D, D), lambda i: (0, 0)),
            pl.BlockSpec((1, D), lambda i: (0, 0)),
            pl.BlockSpec((1, D), lambda i: (0, 0)),
        ],
        out_specs=[
            pl.BlockSpec((BLK, D), lambda i: (i, 0)),
            pl.BlockSpec((BLK, 1), lambda i: (i, 0)),
            pl.BlockSpec((BLK, 1), lambda i: (i, 0)),
            pl.BlockSpec((1, 16), lambda i: (0, 0)),
        ],
        out_shape=[
            jax.ShapeDtypeStruct((n, D), jnp.float32),
            jax.ShapeDtypeStruct((n, 1), jnp.float32),
            jax.ShapeDtypeStruct((n, 1), jnp.float32),
            jax.ShapeDtypeStruct((1, 16), jnp.float32),
        ],
        scratch_shapes=[pltpu.SMEM((2,), jnp.float32)],
    )(x, w1, b1, wg, a_s, a_d)


def _tc_proj2(acc, den, bg, wg, a_s, a_d):
    n = acc.shape[1]
    return pl.pallas_call(
        _tc2_body,
        grid=(n // BLK,),
        in_specs=[
            pl.BlockSpec((2, BLK, D), lambda i: (0, i, 0)),
            pl.BlockSpec((2, BLK, 1), lambda i: (0, i, 0)),
            pl.BlockSpec((1, D), lambda i: (0, 0)),
            pl.BlockSpec((D, D), lambda i: (0, 0)),
            pl.BlockSpec((1, D), lambda i: (0, 0)),
            pl.BlockSpec((1, D), lambda i: (0, 0)),
        ],
        out_specs=[
            pl.BlockSpec((BLK, D), lambda i: (i, 0)),
            pl.BlockSpec((BLK, 1), lambda i: (i, 0)),
            pl.BlockSpec((BLK, 1), lambda i: (i, 0)),
            pl.BlockSpec((1, 16), lambda i: (0, 0)),
        ],
        out_shape=[
            jax.ShapeDtypeStruct((n, D), jnp.float32),
            jax.ShapeDtypeStruct((n, 1), jnp.float32),
            jax.ShapeDtypeStruct((n, 1), jnp.float32),
            jax.ShapeDtypeStruct((1, 16), jnp.float32),
        ],
        scratch_shapes=[pltpu.SMEM((2,), jnp.float32)],
    )(acc, den, bg, wg, a_s, a_d)


def _tc_final(acc, den, bg, w2, b2):
    n = acc.shape[1]
    return pl.pallas_call(
        _tc3_body,
        grid=(n // BLK,),
        in_specs=[
            pl.BlockSpec((2, BLK, D), lambda i: (0, i, 0)),
            pl.BlockSpec((2, BLK, 1), lambda i: (0, i, 0)),
            pl.BlockSpec((1, D), lambda i: (0, 0)),
            pl.BlockSpec((D, C), lambda i: (0, 0)),
            pl.BlockSpec((1, C), lambda i: (0, 0)),
        ],
        out_specs=pl.BlockSpec((BLK, C), lambda i: (i, 0)),
        out_shape=jax.ShapeDtypeStruct((n, C), jnp.float32),
    )(acc, den, bg, w2, b2)


def _sc_gat_body(h_hbm, src_hbm, dst_hbm, asrc_hbm, adst_hbm, g_hbm,
                 acc_out, den_out,
                 srcix, dstix, asr, ads, exv, rowbuf, zbuf, gv,
                 acc_sh, den_sh, sem):
    c = lax.axis_index("c")
    s = lax.axis_index("s")
    w = c * 16 + s
    base = s * RPT

    def _zrow(i, carry):
        for q in range(8):
            rowbuf[i, pl.ds(q * 16, 16)] = jnp.zeros((16,), jnp.float32)
        return carry
    lax.fori_loop(0, CHUNK, _zrow, 0)

    def _zb(i, carry):
        zbuf[pl.ds(i * 16, 16)] = jnp.zeros((16,), jnp.float32)
        return carry
    lax.fori_loop(0, RPT // 16, _zb, 0)

    def _zacc(r, carry):
        pltpu.sync_copy(rowbuf, acc_sh.at[pl.ds(base + r * CHUNK, CHUNK)])
        return carry
    lax.fori_loop(0, RPT // CHUNK, _zacc, 0)
    pltpu.sync_copy(zbuf, den_sh.at[pl.ds(base, RPT)])

    pltpu.sync_copy(src_hbm.at[w], srcix)
    pltpu.sync_copy(dst_hbm.at[w], dstix)
    pltpu.sync_copy(asrc_hbm, asr)
    pltpu.sync_copy(adst_hbm, ads)
    pltpu.sync_copy(g_hbm, gv)
    g = gv[...]

    plsc.subcore_barrier()

    def _chunk1(j, carry):
        for k in range(CHUNK // 16):
            si = srcix[j, pl.ds(k * 16, 16)]
            di = dstix[j, pl.ds(k * 16, 16)]
            t = plsc.load_gather(asr, [si]) + plsc.load_gather(ads, [di])
            al = jnp.where(t >= 0, t, t * 0.2) - g
            exv[pl.ds(j * CHUNK + k * 16, 16)] = jnp.exp(al)
        pltpu.sync_copy(exv.at[pl.ds(j * CHUNK, CHUNK)],
                        den_sh.at[dstix.at[j]], add=True)
        return carry
    lax.fori_loop(0, NCH, _chunk1, 0)

    def _chunk2(j, carry):
        pltpu.async_copy(h_hbm.at[srcix.at[j]], rowbuf, sem).wait()

        def _edge(e, carry2):
            cf = plsc.load_gather(exv, [jnp.full((16,), j * CHUNK + e, jnp.int32)])
            for q in range(8):
                rowbuf[e, pl.ds(q * 16, 16)] = rowbuf[e, pl.ds(q * 16, 16)] * cf
            return carry2
        lax.fori_loop(0, CHUNK, _edge, 0)
        pltpu.sync_copy(rowbuf, acc_sh.at[dstix.at[j]], add=True)
        return carry
    lax.fori_loop(0, NCH, _chunk2, 0)

    plsc.subcore_barrier()

    def _wb(r, carry):
        pltpu.sync_copy(acc_sh.at[pl.ds(base + r * CHUNK, CHUNK)], rowbuf)
        pltpu.sync_copy(rowbuf, acc_out.at[c, pl.ds(base + r * CHUNK, CHUNK)])
        return carry
    lax.fori_loop(0, RPT // CHUNK, _wb, 0)
    pltpu.sync_copy(den_sh.at[pl.ds(base, RPT)], zbuf)
    pltpu.sync_copy(zbuf, den_out.at[c, pl.ds(base, RPT)])


_sc_gat = functools.partial(
    pl.kernel,
    _sc_gat_body,
    out_type=(
        jax.ShapeDtypeStruct((2, N_PAD, D), jnp.float32),
        jax.ShapeDtypeStruct((2, N_PAD), jnp.float32),
    ),
    mesh=plsc.VectorSubcoreMesh(core_axis_name="c", subcore_axis_name="s"),
    scratch_types=[
        pltpu.VMEM((NCH, CHUNK), jnp.int32),
        pltpu.VMEM((NCH, CHUNK), jnp.int32),
        pltpu.VMEM((N,), jnp.float32),
        pltpu.VMEM((N,), jnp.float32),
        pltpu.VMEM((EPT_PAD,), jnp.float32),
        pltpu.VMEM((CHUNK, D), jnp.float32),
        pltpu.VMEM((RPT,), jnp.float32),
        pltpu.VMEM((16,), jnp.float32),
        pltpu.VMEM_SHARED((N_PAD, D), jnp.float32),
        pltpu.VMEM_SHARED((N_PAD,), jnp.float32),
        pltpu.SemaphoreType.DMA,
    ],
)


def kernel(x, edge_index, W1, b1, Wg1, as1, ad1, bg1, Wg2, as2, ad2, bg2, W2, b2):
    src = edge_index[0].reshape(NW, EPT)
    dst = edge_index[1].reshape(NW, EPT)
    src = jnp.pad(src, ((0, 0), (0, EPT_PAD - EPT))).reshape(NW, NCH, CHUNK)
    dst = jnp.pad(dst, ((0, 0), (0, EPT_PAD - EPT)),
                  constant_values=N).reshape(NW, NCH, CHUNK)

    xp = jnp.pad(x, ((0, N_PAD - N), (0, 0)))
    r1 = lambda v: v.reshape(1, -1)

    h1, asrc1, adst1, g1 = _tc_proj1(xp, W1, r1(b1), Wg1, r1(as1), r1(ad1))
    acc1, den1 = _sc_gat()(h1, src, dst, asrc1[:N, 0], adst1[:N, 0],
                           g1.reshape(16))

    h2, asrc2, adst2, g2 = _tc_proj2(acc1, den1[..., None], r1(bg1), Wg2,
                                     r1(as2), r1(ad2))
    acc2, den2 = _sc_gat()(h2, src, dst, asrc2[:N, 0], adst2[:N, 0],
                           g2.reshape(16))

    out = _tc_final(acc2, den2[..., None], r1(bg2), W2, b2.reshape(1, C))
    return out[:N]


# speedup vs baseline: 16.9463x; 16.9463x over previous
import functools

import jax
import jax.numpy as jnp
from jax import lax
from jax.experimental import pallas as pl
from jax.experimental.pallas import tpu as pltpu
from jax.experimental.pallas import tpu_sc as plsc

N = 10000
D = 128
C = 40
E = 320000

NW = 32
EPT = E // NW
CHUNK = 128
NCH = (EPT + CHUNK - 1) // CHUNK
EPT_PAD = NCH * CHUNK
N_PAD = 10240
RPT = N_PAD // 16
BLK = 512


def _proj_tail(h, as_ref, ad_ref, asrc_ref, adst_ref, g_ref, mx_ref):
    i = pl.program_id(0)
    asrc = jnp.sum(h * as_ref[...], axis=1, keepdims=True)
    adst = jnp.sum(h * ad_ref[...], axis=1, keepdims=True)
    asrc_ref[...] = asrc
    adst_ref[...] = adst

    @pl.when(i == 0)
    def _():
        mx_ref[0] = -jnp.inf
        mx_ref[1] = -jnp.inf

    mx_ref[0] = jnp.maximum(mx_ref[0], jnp.max(asrc))
    mx_ref[1] = jnp.maximum(mx_ref[1], jnp.max(adst))

    @pl.when(i == pl.num_programs(0) - 1)
    def _():
        s = mx_ref[0] + mx_ref[1]
        g = jnp.where(s >= 0, s, 0.2 * s)
        g_ref[...] = jnp.full((1, 16), g, jnp.float32)


def _tc1_body(x_ref, w1_ref, b1_ref, wg_ref, as_ref, ad_ref,
              h_ref, asrc_ref, adst_ref, g_ref, mx_ref):
    h0 = jnp.maximum(jnp.dot(x_ref[...], w1_ref[...],
                             preferred_element_type=jnp.float32) + b1_ref[...], 0.0)
    h = jnp.dot(h0, wg_ref[...], preferred_element_type=jnp.float32)
    h_ref[...] = h
    _proj_tail(h, as_ref, ad_ref, asrc_ref, adst_ref, g_ref, mx_ref)


def _tc2_body(acc_ref, den_ref, bg_ref, wg_ref, as_ref, ad_ref,
              h_ref, asrc_ref, adst_ref, g_ref, mx_ref):
    agg = acc_ref[0] + acc_ref[1]
    dn = den_ref[0] + den_ref[1]
    out = agg / (dn + 1e-16) + bg_ref[...]
    h1 = jnp.maximum(out, 0.0)
    h = jnp.dot(h1, wg_ref[...], preferred_element_type=jnp.float32)
    h_ref[...] = h
    _proj_tail(h, as_ref, ad_ref, asrc_ref, adst_ref, g_ref, mx_ref)


def _tc3_body(acc_ref, den_ref, bg_ref, w2_ref, b2_ref, o_ref):
    agg = acc_ref[0] + acc_ref[1]
    dn = den_ref[0] + den_ref[1]
    out = agg / (dn + 1e-16) + bg_ref[...]
    logits = jnp.dot(out, w2_ref[...],
                     preferred_element_type=jnp.float32) + b2_ref[...]
    m = jnp.max(logits, axis=1, keepdims=True)
    ls = logits - m
    o_ref[...] = ls - jnp.log(jnp.sum(jnp.exp(ls), axis=1, keepdims=True))


def _tc_proj1(x, w1, b1, wg, a_s, a_d):
    n = x.shape[0]
    return pl.pallas_call(
        _tc1_body,
        grid=(n // BLK,),
        in_specs=[
            pl.BlockSpec((BLK, D), lambda i: (i, 0)),
            pl.BlockSpec((D, D), lambda i: (0, 0)),
            pl.BlockSpec((1, D), lambda i: (0, 0)),
            pl.BlockSpec((D, D), lambda i: (0, 0)),
            pl.BlockSpec((1, D), lambda i: (0, 0)),
            pl.BlockSpec((1, D), lambda i: (0, 0)),
        ],
        out_specs=[
            pl.BlockSpec((BLK, D), lambda i: (i, 0)),
            pl.BlockSpec((BLK, 1), lambda i: (i, 0)),
            pl.BlockSpec((BLK, 1), lambda i: (i, 0)),
            pl.BlockSpec((1, 16), lambda i: (0, 0)),
        ],
        out_shape=[
            jax.ShapeDtypeStruct((n, D), jnp.float32),
            jax.ShapeDtypeStruct((n, 1), jnp.float32),
            jax.ShapeDtypeStruct((n, 1), jnp.float32),
            jax.ShapeDtypeStruct((1, 16), jnp.float32),
        ],
        scratch_shapes=[pltpu.SMEM((2,), jnp.float32)],
    )(x, w1, b1, wg, a_s, a_d)


def _tc_proj2(acc, den, bg, wg, a_s, a_d):
    n = acc.shape[1]
    return pl.pallas_call(
        _tc2_body,
        grid=(n // BLK,),
        in_specs=[
            pl.BlockSpec((2, BLK, D), lambda i: (0, i, 0)),
            pl.BlockSpec((2, BLK, 1), lambda i: (0, i, 0)),
            pl.BlockSpec((1, D), lambda i: (0, 0)),
            pl.BlockSpec((D, D), lambda i: (0, 0)),
            pl.BlockSpec((1, D), lambda i: (0, 0)),
            pl.BlockSpec((1, D), lambda i: (0, 0)),
        ],
        out_specs=[
            pl.BlockSpec((BLK, D), lambda i: (i, 0)),
            pl.BlockSpec((BLK, 1), lambda i: (i, 0)),
            pl.BlockSpec((BLK, 1), lambda i: (i, 0)),
            pl.BlockSpec((1, 16), lambda i: (0, 0)),
        ],
        out_shape=[
            jax.ShapeDtypeStruct((n, D), jnp.float32),
            jax.ShapeDtypeStruct((n, 1), jnp.float32),
            jax.ShapeDtypeStruct((n, 1), jnp.float32),
            jax.ShapeDtypeStruct((1, 16), jnp.float32),
        ],
        scratch_shapes=[pltpu.SMEM((2,), jnp.float32)],
    )(acc, den, bg, wg, a_s, a_d)


def _tc_final(acc, den, bg, w2, b2):
    n = acc.shape[1]
    return pl.pallas_call(
        _tc3_body,
        grid=(n // BLK,),
        in_specs=[
            pl.BlockSpec((2, BLK, D), lambda i: (0, i, 0)),
            pl.BlockSpec((2, BLK, 1), lambda i: (0, i, 0)),
            pl.BlockSpec((1, D), lambda i: (0, 0)),
            pl.BlockSpec((D, C), lambda i: (0, 0)),
            pl.BlockSpec((1, C), lambda i: (0, 0)),
        ],
        out_specs=pl.BlockSpec((BLK, C), lambda i: (i, 0)),
        out_shape=jax.ShapeDtypeStruct((n, C), jnp.float32),
    )(acc, den, bg, w2, b2)


def _sc_gat_body(h_hbm, src_hbm, dst_hbm, asrc_hbm, adst_hbm, g_hbm,
                 acc_out, den_out,
                 srcix, dstix, abuf, bbuf, exv, rowbuf, zbuf, gv,
                 acc_sh, den_sh, sem):
    c = lax.axis_index("c")
    s = lax.axis_index("s")
    w = c * 16 + s
    base = s * RPT

    def _zrow(i, carry):
        for q in range(8):
            rowbuf[i, pl.ds(q * 16, 16)] = jnp.zeros((16,), jnp.float32)
        return carry
    lax.fori_loop(0, CHUNK, _zrow, 0)

    def _zb(i, carry):
        zbuf[pl.ds(i * 16, 16)] = jnp.zeros((16,), jnp.float32)
        return carry
    lax.fori_loop(0, RPT // 16, _zb, 0)

    def _zacc(r, carry):
        pltpu.sync_copy(rowbuf, acc_sh.at[pl.ds(base + r * CHUNK, CHUNK)])
        return carry
    lax.fori_loop(0, RPT // CHUNK, _zacc, 0)
    pltpu.sync_copy(zbuf, den_sh.at[pl.ds(base, RPT)])

    pltpu.sync_copy(src_hbm.at[w], srcix)
    pltpu.sync_copy(dst_hbm.at[w], dstix)
    pltpu.sync_copy(g_hbm, gv)
    g = gv[...]

    plsc.subcore_barrier()

    def _chunk1(j, carry):
        pltpu.async_copy(asrc_hbm.at[srcix.at[j]], abuf, sem).wait()
        pltpu.async_copy(adst_hbm.at[dstix.at[j]], bbuf, sem).wait()
        for k in range(CHUNK // 16):
            t = abuf[pl.ds(k * 16, 16)] + bbuf[pl.ds(k * 16, 16)]
            al = jnp.where(t >= 0, t, t * 0.2) - g
            exv[pl.ds(j * CHUNK + k * 16, 16)] = jnp.exp(al)
        pltpu.sync_copy(exv.at[pl.ds(j * CHUNK, CHUNK)],
                        den_sh.at[dstix.at[j]], add=True)
        return carry
    lax.fori_loop(0, NCH, _chunk1, 0)

    def _chunk2(j, carry):
        pltpu.async_copy(h_hbm.at[srcix.at[j]], rowbuf, sem).wait()

        def _edge(e, carry2):
            cf = plsc.load_gather(exv, [jnp.full((16,), j * CHUNK + e, jnp.int32)])
            for q in range(8):
                rowbuf[e, pl.ds(q * 16, 16)] = rowbuf[e, pl.ds(q * 16, 16)] * cf
            return carry2
        lax.fori_loop(0, CHUNK, _edge, 0)
        pltpu.sync_copy(rowbuf, acc_sh.at[dstix.at[j]], add=True)
        return carry
    lax.fori_loop(0, NCH, _chunk2, 0)

    plsc.subcore_barrier()

    def _wb(r, carry):
        pltpu.sync_copy(acc_sh.at[pl.ds(base + r * CHUNK, CHUNK)], rowbuf)
        pltpu.sync_copy(rowbuf, acc_out.at[c, pl.ds(base + r * CHUNK, CHUNK)])
        return carry
    lax.fori_loop(0, RPT // CHUNK, _wb, 0)
    pltpu.sync_copy(den_sh.at[pl.ds(base, RPT)], zbuf)
    pltpu.sync_copy(zbuf, den_out.at[c, pl.ds(base, RPT)])


@functools.lru_cache(maxsize=1)
def _sc_gat():
    return pl.kernel(
        _sc_gat_body,
        out_type=(
            jax.ShapeDtypeStruct((2, N_PAD, D), jnp.float32),
            jax.ShapeDtypeStruct((2, N_PAD), jnp.float32),
        ),
        mesh=plsc.VectorSubcoreMesh(core_axis_name="c", subcore_axis_name="s",
                                    num_cores=2, num_subcores=16),
        scratch_types=[
        pltpu.VMEM((NCH, CHUNK), jnp.int32),
        pltpu.VMEM((NCH, CHUNK), jnp.int32),
        pltpu.VMEM((CHUNK,), jnp.float32),
        pltpu.VMEM((CHUNK,), jnp.float32),
        pltpu.VMEM((EPT_PAD,), jnp.float32),
        pltpu.VMEM((CHUNK, D), jnp.float32),
        pltpu.VMEM((RPT,), jnp.float32),
        pltpu.VMEM((16,), jnp.float32),
            pltpu.VMEM_SHARED((N_PAD, D), jnp.float32),
            pltpu.VMEM_SHARED((N_PAD,), jnp.float32),
            pltpu.SemaphoreType.DMA,
        ],
        compiler_params=pltpu.CompilerParams(needs_layout_passes=False),
    )


def kernel(x, edge_index, W1, b1, Wg1, as1, ad1, bg1, Wg2, as2, ad2, bg2, W2, b2):
    src = edge_index[0].reshape(NW, EPT)
    dst = edge_index[1].reshape(NW, EPT)
    src = jnp.pad(src, ((0, 0), (0, EPT_PAD - EPT))).reshape(NW, NCH, CHUNK)
    dst = jnp.pad(dst, ((0, 0), (0, EPT_PAD - EPT)),
                  constant_values=N).reshape(NW, NCH, CHUNK)

    xp = jnp.pad(x, ((0, N_PAD - N), (0, 0)))
    r1 = lambda v: v.reshape(1, -1)

    h1, asrc1, adst1, g1 = _tc_proj1(xp, W1, r1(b1), Wg1, r1(as1), r1(ad1))
    acc1, den1 = _sc_gat()(h1, src, dst, asrc1[:N, 0], adst1[:N, 0],
                           g1.reshape(16))

    h2, asrc2, adst2, g2 = _tc_proj2(acc1, den1[..., None], r1(bg1), Wg2,
                                     r1(as2), r1(ad2))
    acc2, den2 = _sc_gat()(h2, src, dst, asrc2[:N, 0], adst2[:N, 0],
                           g2.reshape(16))

    out = _tc_final(acc2, den2[..., None], r1(bg2), W2, b2.reshape(1, C))
    return out[:N]


# speedup vs baseline: 21.5841x; 1.2737x over previous
import functools

import jax
import jax.numpy as jnp
from jax import lax
from jax.experimental import pallas as pl
from jax.experimental.pallas import tpu as pltpu
from jax.experimental.pallas import tpu_sc as plsc

N = 10000
D = 128
C = 40
E = 320000

NW = 32
EPT = E // NW
CHUNK = 64
NCH = 2 * ((EPT + 2 * CHUNK - 1) // (2 * CHUNK))
EPT_PAD = NCH * CHUNK
NCHP = NCH // 2
N_PAD = 10112
RPT = N_PAD // 16
BLK = 632


def _proj_tail(h, as_ref, ad_ref, asrc_ref, adst_ref, g_ref, mx_ref):
    i = pl.program_id(0)
    asrc = jnp.sum(h * as_ref[...], axis=1, keepdims=True)
    adst = jnp.sum(h * ad_ref[...], axis=1, keepdims=True)
    asrc_ref[...] = asrc
    adst_ref[...] = adst

    @pl.when(i == 0)
    def _():
        mx_ref[0] = -jnp.inf
        mx_ref[1] = -jnp.inf

    mx_ref[0] = jnp.maximum(mx_ref[0], jnp.max(asrc))
    mx_ref[1] = jnp.maximum(mx_ref[1], jnp.max(adst))

    @pl.when(i == pl.num_programs(0) - 1)
    def _():
        s = mx_ref[0] + mx_ref[1]
        g = jnp.where(s >= 0, s, 0.2 * s)
        g_ref[...] = jnp.full((1, 16), g, jnp.float32)


def _tc1_body(x_ref, w1_ref, b1_ref, wg_ref, as_ref, ad_ref,
              h_ref, asrc_ref, adst_ref, g_ref, mx_ref):
    h0 = jnp.maximum(jnp.dot(x_ref[...], w1_ref[...],
                             preferred_element_type=jnp.float32) + b1_ref[...], 0.0)
    h = jnp.dot(h0, wg_ref[...], preferred_element_type=jnp.float32)
    h_ref[...] = h
    _proj_tail(h, as_ref, ad_ref, asrc_ref, adst_ref, g_ref, mx_ref)


def _tc2_body(acc_ref, den_ref, bg_ref, wg_ref, as_ref, ad_ref,
              h_ref, asrc_ref, adst_ref, g_ref, mx_ref):
    agg = acc_ref[0] + acc_ref[1]
    dn = den_ref[0] + den_ref[1]
    out = agg / (dn + 1e-16) + bg_ref[...]
    h1 = jnp.maximum(out, 0.0)
    h = jnp.dot(h1, wg_ref[...], preferred_element_type=jnp.float32)
    h_ref[...] = h
    _proj_tail(h, as_ref, ad_ref, asrc_ref, adst_ref, g_ref, mx_ref)


def _tc3_body(acc_ref, den_ref, bg_ref, w2_ref, b2_ref, o_ref):
    agg = acc_ref[0] + acc_ref[1]
    dn = den_ref[0] + den_ref[1]
    out = agg / (dn + 1e-16) + bg_ref[...]
    logits = jnp.dot(out, w2_ref[...],
                     preferred_element_type=jnp.float32) + b2_ref[...]
    m = jnp.max(logits, axis=1, keepdims=True)
    ls = logits - m
    o_ref[...] = ls - jnp.log(jnp.sum(jnp.exp(ls), axis=1, keepdims=True))


def _tc_proj1(x, w1, b1, wg, a_s, a_d):
    n = x.shape[0]
    return pl.pallas_call(
        _tc1_body,
        grid=(n // BLK,),
        in_specs=[
            pl.BlockSpec((BLK, D), lambda i: (i, 0)),
            pl.BlockSpec((D, D), lambda i: (0, 0)),
            pl.BlockSpec((1, D), lambda i: (0, 0)),
            pl.BlockSpec((D, D), lambda i: (0, 0)),
            pl.BlockSpec((1, D), lambda i: (0, 0)),
            pl.BlockSpec((1, D), lambda i: (0, 0)),
        ],
        out_specs=[
            pl.BlockSpec((BLK, D), lambda i: (i, 0)),
            pl.BlockSpec((BLK, 1), lambda i: (i, 0)),
            pl.BlockSpec((BLK, 1), lambda i: (i, 0)),
            pl.BlockSpec((1, 16), lambda i: (0, 0)),
        ],
        out_shape=[
            jax.ShapeDtypeStruct((n, D), jnp.float32),
            jax.ShapeDtypeStruct((n, 1), jnp.float32),
            jax.ShapeDtypeStruct((n, 1), jnp.float32),
            jax.ShapeDtypeStruct((1, 16), jnp.float32),
        ],
        scratch_shapes=[pltpu.SMEM((2,), jnp.float32)],
    )(x, w1, b1, wg, a_s, a_d)


def _tc_proj2(acc, den, bg, wg, a_s, a_d):
    n = acc.shape[1]
    return pl.pallas_call(
        _tc2_body,
        grid=(n // BLK,),
        in_specs=[
            pl.BlockSpec((2, BLK, D), lambda i: (0, i, 0)),
            pl.BlockSpec((2, BLK, 1), lambda i: (0, i, 0)),
            pl.BlockSpec((1, D), lambda i: (0, 0)),
            pl.BlockSpec((D, D), lambda i: (0, 0)),
            pl.BlockSpec((1, D), lambda i: (0, 0)),
            pl.BlockSpec((1, D), lambda i: (0, 0)),
        ],
        out_specs=[
            pl.BlockSpec((BLK, D), lambda i: (i, 0)),
            pl.BlockSpec((BLK, 1), lambda i: (i, 0)),
            pl.BlockSpec((BLK, 1), lambda i: (i, 0)),
            pl.BlockSpec((1, 16), lambda i: (0, 0)),
        ],
        out_shape=[
            jax.ShapeDtypeStruct((n, D), jnp.float32),
            jax.ShapeDtypeStruct((n, 1), jnp.float32),
            jax.ShapeDtypeStruct((n, 1), jnp.float32),
            jax.ShapeDtypeStruct((1, 16), jnp.float32),
        ],
        scratch_shapes=[pltpu.SMEM((2,), jnp.float32)],
    )(acc, den, bg, wg, a_s, a_d)


def _tc_final(acc, den, bg, w2, b2):
    n = acc.shape[1]
    return pl.pallas_call(
        _tc3_body,
        grid=(n // BLK,),
        in_specs=[
            pl.BlockSpec((2, BLK, D), lambda i: (0, i, 0)),
            pl.BlockSpec((2, BLK, 1), lambda i: (0, i, 0)),
            pl.BlockSpec((1, D), lambda i: (0, 0)),
            pl.BlockSpec((D, C), lambda i: (0, 0)),
            pl.BlockSpec((1, C), lambda i: (0, 0)),
        ],
        out_specs=pl.BlockSpec((BLK, C), lambda i: (i, 0)),
        out_shape=jax.ShapeDtypeStruct((n, C), jnp.float32),
    )(acc, den, bg, w2, b2)


def _sc_gat_body(h_hbm, src_hbm, dst_hbm, asrc_hbm, adst_hbm, g_hbm,
                 acc_out, den_out,
                 srcix, dstix, abuf, bbuf, exbuf, rowbuf, gv,
                 acc_sh, den_sh,
                 sem_a0, sem_a1, sem_b0, sem_b1, sem_r0, sem_r1,
                 sem_d0, sem_d1, sem_s0, sem_s1):
    cid = lax.axis_index("c")
    s = lax.axis_index("s")
    w = cid * 16 + s
    base = s * RPT
    sa = (sem_a0, sem_a1)
    sb = (sem_b0, sem_b1)
    sr = (sem_r0, sem_r1)
    sd = (sem_d0, sem_d1)
    ss = (sem_s0, sem_s1)

    def _zrow(i, carry):
        for q in range(8):
            rowbuf[i, pl.ds(q * 16, 16)] = jnp.zeros((16,), jnp.float32)
        return carry
    lax.fori_loop(0, 2 * CHUNK, _zrow, 0)

    for r in range(4):
        pltpu.sync_copy(rowbuf, acc_sh.at[pl.ds(base + r * 128, 128)])
        pltpu.sync_copy(rowbuf.at[0], den_sh.at[pl.ds(base + r * 128, 128)])
    pltpu.sync_copy(rowbuf.at[pl.ds(0, RPT - 512)],
                    acc_sh.at[pl.ds(base + 512, RPT - 512)])
    pltpu.sync_copy(rowbuf.at[0, pl.ds(0, RPT - 512)],
                    den_sh.at[pl.ds(base + 512, RPT - 512)])

    pltpu.sync_copy(g_hbm, gv)
    g = gv[...]

    plsc.subcore_barrier()

    def _slot(buf, b):
        return buf.at[pl.ds(b * CHUNK, CHUNK)]

    def _fire_gathers(cc, b):
        pltpu.async_copy(asrc_hbm.at[srcix.at[cc]], _slot(abuf, b), sa[b])
        pltpu.async_copy(adst_hbm.at[dstix.at[cc]], _slot(bbuf, b), sb[b])
        pltpu.async_copy(h_hbm.at[srcix.at[cc]],
                         rowbuf.at[pl.ds(b * CHUNK, CHUNK)], sr[b])

    def _wait_alpha(cc, b):
        pltpu.make_async_copy(asrc_hbm.at[srcix.at[cc]], _slot(abuf, b),
                              sa[b]).wait()
        pltpu.make_async_copy(adst_hbm.at[dstix.at[cc]], _slot(bbuf, b),
                              sb[b]).wait()

    def _wait_rows(cc, b):
        pltpu.make_async_copy(h_hbm.at[srcix.at[cc]],
                              rowbuf.at[pl.ds(b * CHUNK, CHUNK)], sr[b]).wait()

    def _fire_den(cc, b):
        pltpu.async_copy(_slot(exbuf, b), den_sh.at[dstix.at[cc]], sd[b],
                         add=True)

    def _wait_den(cc, b):
        pltpu.make_async_copy(_slot(exbuf, b), den_sh.at[dstix.at[cc]],
                              sd[b]).wait()

    def _fire_rs(cc, b):
        pltpu.async_copy(rowbuf.at[pl.ds(b * CHUNK, CHUNK)],
                         acc_sh.at[dstix.at[cc]], ss[b], add=True)

    def _wait_rs(cc, b):
        pltpu.make_async_copy(rowbuf.at[pl.ds(b * CHUNK, CHUNK)],
                              acc_sh.at[dstix.at[cc]], ss[b]).wait()

    def _compute_ex(b):
        for k in range(CHUNK // 16):
            off = b * CHUNK + k * 16
            t = abuf[pl.ds(off, 16)] + bbuf[pl.ds(off, 16)]
            al = jnp.where(t >= 0, t, t * 0.2) - g
            exbuf[pl.ds(off, 16)] = jnp.exp(al)

    def _scale(b):
        def _edge(e, carry):
            r = b * CHUNK + e
            cf = plsc.load_gather(exbuf, [jnp.full((16,), r, jnp.int32)])
            for q in range(8):
                rowbuf[r, pl.ds(q * 16, 16)] = rowbuf[r, pl.ds(q * 16, 16)] * cf
            return carry
        lax.fori_loop(0, CHUNK, _edge, 0)

    def _step(cc, b, kind):
        if kind >= 2:
            _wait_den(cc - 2, b)
        _wait_alpha(cc, b)
        _compute_ex(b)
        _fire_den(cc, b)
        _wait_rows(cc, b)
        _scale(b)
        _fire_rs(cc, b)
        if kind >= 1:
            _wait_rs(cc - 1, 1 - b)
        if kind <= 2:
            _fire_gathers(cc + 1, 1 - b)

    for hf in range(2):
        pltpu.sync_copy(src_hbm.at[w, hf], srcix)
        pltpu.sync_copy(dst_hbm.at[w, hf], dstix)

        _fire_gathers(0, 0)
        _step(0, 0, 0)
        _step(1, 1, 1)

        def _pass_body(t, carry):
            c0 = 2 * t
            _step(c0, 0, 2)
            _step(c0 + 1, 1, 2)
            return carry
        lax.fori_loop(1, (NCHP - 1) // 2, _pass_body, 0)

        _step(NCHP - 1, 0, 3)
        _wait_den(NCHP - 2, 1)
        _wait_den(NCHP - 1, 0)
        _wait_rs(NCHP - 1, 0)

    plsc.subcore_barrier()

    obase = cid * N_PAD + base
    for r in range(4):
        pltpu.sync_copy(acc_sh.at[pl.ds(base + r * 128, 128)], rowbuf)
        pltpu.sync_copy(rowbuf, acc_out.at[pl.ds(obase + r * 128, 128)])
        pltpu.sync_copy(den_sh.at[pl.ds(base + r * 128, 128)], abuf)
        pltpu.sync_copy(abuf, den_out.at[pl.ds(obase + r * 128, 128)])
    pltpu.sync_copy(acc_sh.at[pl.ds(base + 512, RPT - 512)],
                    rowbuf.at[pl.ds(0, RPT - 512)])
    pltpu.sync_copy(rowbuf.at[pl.ds(0, RPT - 512)],
                    acc_out.at[pl.ds(obase + 512, RPT - 512)])
    pltpu.sync_copy(den_sh.at[pl.ds(base + 512, RPT - 512)],
                    abuf.at[pl.ds(0, RPT - 512)])
    pltpu.sync_copy(abuf.at[pl.ds(0, RPT - 512)],
                    den_out.at[pl.ds(obase + 512, RPT - 512)])


@functools.lru_cache(maxsize=1)
def _sc_gat():
    return pl.kernel(
        _sc_gat_body,
        out_type=(
            jax.ShapeDtypeStruct((2 * N_PAD, D), jnp.float32),
            jax.ShapeDtypeStruct((2 * N_PAD,), jnp.float32),
        ),
        mesh=plsc.VectorSubcoreMesh(core_axis_name="c", subcore_axis_name="s",
                                    num_cores=2, num_subcores=16),
        scratch_types=[
            pltpu.VMEM((NCHP, CHUNK), jnp.int32),
            pltpu.VMEM((NCHP, CHUNK), jnp.int32),
            pltpu.VMEM((2 * CHUNK,), jnp.float32),
            pltpu.VMEM((2 * CHUNK,), jnp.float32),
            pltpu.VMEM((2 * CHUNK,), jnp.float32),
            pltpu.VMEM((2 * CHUNK, D), jnp.float32),
            pltpu.VMEM((16,), jnp.float32),
            pltpu.VMEM_SHARED((N_PAD, D), jnp.float32),
            pltpu.VMEM_SHARED((N_PAD,), jnp.float32),
            pltpu.SemaphoreType.DMA,
            pltpu.SemaphoreType.DMA,
            pltpu.SemaphoreType.DMA,
            pltpu.SemaphoreType.DMA,
            pltpu.SemaphoreType.DMA,
            pltpu.SemaphoreType.DMA,
            pltpu.SemaphoreType.DMA,
            pltpu.SemaphoreType.DMA,
            pltpu.SemaphoreType.DMA,
            pltpu.SemaphoreType.DMA,
        ],
        compiler_params=pltpu.CompilerParams(needs_layout_passes=False),
    )


def kernel(x, edge_index, W1, b1, Wg1, as1, ad1, bg1, Wg2, as2, ad2, bg2, W2, b2):
    src = edge_index[0].reshape(NW, EPT)
    dst = edge_index[1].reshape(NW, EPT)
    src = jnp.pad(src, ((0, 0), (0, EPT_PAD - EPT))).reshape(NW, 2, NCHP, CHUNK)
    dst = jnp.pad(dst, ((0, 0), (0, EPT_PAD - EPT)),
                  constant_values=N).reshape(NW, 2, NCHP, CHUNK)

    xp = jnp.pad(x, ((0, N_PAD - N), (0, 0)))
    r1 = lambda v: v.reshape(1, -1)

    h1, asrc1, adst1, g1 = _tc_proj1(xp, W1, r1(b1), Wg1, r1(as1), r1(ad1))
    acc1, den1 = _sc_gat()(h1, src, dst, asrc1[:N, 0], adst1[:N, 0],
                           g1.reshape(16))
    acc1 = acc1.reshape(2, N_PAD, D)
    den1 = den1.reshape(2, N_PAD)

    h2, asrc2, adst2, g2 = _tc_proj2(acc1, den1[..., None], r1(bg1), Wg2,
                                     r1(as2), r1(ad2))
    acc2, den2 = _sc_gat()(h2, src, dst, asrc2[:N, 0], adst2[:N, 0],
                           g2.reshape(16))
    acc2 = acc2.reshape(2, N_PAD, D)
    den2 = den2.reshape(2, N_PAD)

    out = _tc_final(acc2, den2[..., None], r1(bg2), W2, b2.reshape(1, C))
    return out[:N]


# speedup vs baseline: 26.8374x; 1.2434x over previous
import functools

import jax
import jax.numpy as jnp
from jax import lax
from jax.experimental import pallas as pl
from jax.experimental.pallas import tpu as pltpu
from jax.experimental.pallas import tpu_sc as plsc

N = 10000
D = 128
C = 40
E = 320000

NW = 32
EPT = E // NW
CHUNK = 64
NCH = 2 * ((EPT + 2 * CHUNK - 1) // (2 * CHUNK))
EPT_PAD = NCH * CHUNK
NCHP = NCH // 2
N_PAD = 10112
RPT = N_PAD // 16
BLK = 632


def _proj_tail(h, as_ref, ad_ref, asrc_ref, adst_ref, g_ref, mx_ref):
    i = pl.program_id(0)
    asrc = jnp.sum(h * as_ref[...], axis=1, keepdims=True)
    adst = jnp.sum(h * ad_ref[...], axis=1, keepdims=True)
    asrc_ref[...] = asrc
    adst_ref[...] = adst

    @pl.when(i == 0)
    def _():
        mx_ref[0] = -jnp.inf
        mx_ref[1] = -jnp.inf

    mx_ref[0] = jnp.maximum(mx_ref[0], jnp.max(asrc))
    mx_ref[1] = jnp.maximum(mx_ref[1], jnp.max(adst))

    @pl.when(i == pl.num_programs(0) - 1)
    def _():
        s = mx_ref[0] + mx_ref[1]
        g = jnp.where(s >= 0, s, 0.2 * s)
        g_ref[...] = jnp.full((1, 16), g, jnp.float32)


def _tc1_body(x_ref, w1_ref, b1_ref, wg_ref, as_ref, ad_ref,
              h_ref, asrc_ref, adst_ref, g_ref, mx_ref):
    h0 = jnp.maximum(jnp.dot(x_ref[...], w1_ref[...],
                             preferred_element_type=jnp.float32) + b1_ref[...], 0.0)
    h = jnp.dot(h0, wg_ref[...], preferred_element_type=jnp.float32)
    h_ref[...] = h
    _proj_tail(h, as_ref, ad_ref, asrc_ref, adst_ref, g_ref, mx_ref)


def _tc2_body(acc_ref, den_ref, bg_ref, wg_ref, as_ref, ad_ref,
              h_ref, asrc_ref, adst_ref, g_ref, mx_ref):
    agg = acc_ref[0] + acc_ref[1]
    dn = den_ref[0] + den_ref[1]
    out = agg / (dn + 1e-16) + bg_ref[...]
    h1 = jnp.maximum(out, 0.0)
    h = jnp.dot(h1, wg_ref[...], preferred_element_type=jnp.float32)
    h_ref[...] = h
    _proj_tail(h, as_ref, ad_ref, asrc_ref, adst_ref, g_ref, mx_ref)


def _tc3_body(acc_ref, den_ref, bg_ref, w2_ref, b2_ref, o_ref):
    agg = acc_ref[0] + acc_ref[1]
    dn = den_ref[0] + den_ref[1]
    out = agg / (dn + 1e-16) + bg_ref[...]
    logits = jnp.dot(out, w2_ref[...],
                     preferred_element_type=jnp.float32) + b2_ref[...]
    m = jnp.max(logits, axis=1, keepdims=True)
    ls = logits - m
    o_ref[...] = ls - jnp.log(jnp.sum(jnp.exp(ls), axis=1, keepdims=True))


def _tc_proj1(x, w1, b1, wg, a_s, a_d):
    n = x.shape[0]
    return pl.pallas_call(
        _tc1_body,
        grid=(n // BLK,),
        in_specs=[
            pl.BlockSpec((BLK, D), lambda i: (i, 0)),
            pl.BlockSpec((D, D), lambda i: (0, 0)),
            pl.BlockSpec((1, D), lambda i: (0, 0)),
            pl.BlockSpec((D, D), lambda i: (0, 0)),
            pl.BlockSpec((1, D), lambda i: (0, 0)),
            pl.BlockSpec((1, D), lambda i: (0, 0)),
        ],
        out_specs=[
            pl.BlockSpec((BLK, D), lambda i: (i, 0)),
            pl.BlockSpec((BLK, 1), lambda i: (i, 0)),
            pl.BlockSpec((BLK, 1), lambda i: (i, 0)),
            pl.BlockSpec((1, 16), lambda i: (0, 0)),
        ],
        out_shape=[
            jax.ShapeDtypeStruct((n, D), jnp.float32),
            jax.ShapeDtypeStruct((n, 1), jnp.float32),
            jax.ShapeDtypeStruct((n, 1), jnp.float32),
            jax.ShapeDtypeStruct((1, 16), jnp.float32),
        ],
        scratch_shapes=[pltpu.SMEM((2,), jnp.float32)],
    )(x, w1, b1, wg, a_s, a_d)


def _tc_proj2(acc, den, bg, wg, a_s, a_d):
    n = acc.shape[1]
    return pl.pallas_call(
        _tc2_body,
        grid=(n // BLK,),
        in_specs=[
            pl.BlockSpec((2, BLK, D), lambda i: (0, i, 0)),
            pl.BlockSpec((2, BLK, 1), lambda i: (0, i, 0)),
            pl.BlockSpec((1, D), lambda i: (0, 0)),
            pl.BlockSpec((D, D), lambda i: (0, 0)),
            pl.BlockSpec((1, D), lambda i: (0, 0)),
            pl.BlockSpec((1, D), lambda i: (0, 0)),
        ],
        out_specs=[
            pl.BlockSpec((BLK, D), lambda i: (i, 0)),
            pl.BlockSpec((BLK, 1), lambda i: (i, 0)),
            pl.BlockSpec((BLK, 1), lambda i: (i, 0)),
            pl.BlockSpec((1, 16), lambda i: (0, 0)),
        ],
        out_shape=[
            jax.ShapeDtypeStruct((n, D), jnp.float32),
            jax.ShapeDtypeStruct((n, 1), jnp.float32),
            jax.ShapeDtypeStruct((n, 1), jnp.float32),
            jax.ShapeDtypeStruct((1, 16), jnp.float32),
        ],
        scratch_shapes=[pltpu.SMEM((2,), jnp.float32)],
    )(acc, den, bg, wg, a_s, a_d)


def _tc_final(acc, den, bg, w2, b2):
    n = acc.shape[1]
    return pl.pallas_call(
        _tc3_body,
        grid=(n // BLK,),
        in_specs=[
            pl.BlockSpec((2, BLK, D), lambda i: (0, i, 0)),
            pl.BlockSpec((2, BLK, 1), lambda i: (0, i, 0)),
            pl.BlockSpec((1, D), lambda i: (0, 0)),
            pl.BlockSpec((D, C), lambda i: (0, 0)),
            pl.BlockSpec((1, C), lambda i: (0, 0)),
        ],
        out_specs=pl.BlockSpec((BLK, C), lambda i: (i, 0)),
        out_shape=jax.ShapeDtypeStruct((n, C), jnp.float32),
    )(acc, den, bg, w2, b2)


def _sc_gat_body(h_hbm, src_hbm, dst_hbm, asrc_hbm, adst_hbm, g_hbm,
                 acc_out, den_out,
                 srcix, dstix, abuf, bbuf, exbuf, rowbuf, gv,
                 acc_sh, den_sh,
                 sem_a0, sem_a1, sem_b0, sem_b1, sem_r0, sem_r1,
                 sem_d0, sem_d1, sem_s0, sem_s1):
    cid = lax.axis_index("c")
    s = lax.axis_index("s")
    w = cid * 16 + s
    base = s * RPT
    sa = (sem_a0, sem_a1)
    sb = (sem_b0, sem_b1)
    sr = (sem_r0, sem_r1)
    sd = (sem_d0, sem_d1)
    ss = (sem_s0, sem_s1)

    def _zrow(i, carry):
        for q in range(8):
            rowbuf[i, pl.ds(q * 16, 16)] = jnp.zeros((16,), jnp.float32)
        return carry
    lax.fori_loop(0, 2 * CHUNK, _zrow, 0)

    for r in range(4):
        pltpu.sync_copy(rowbuf, acc_sh.at[pl.ds(base + r * 128, 128)])
        pltpu.sync_copy(rowbuf.at[0], den_sh.at[pl.ds(base + r * 128, 128)])
    pltpu.sync_copy(rowbuf.at[pl.ds(0, RPT - 512)],
                    acc_sh.at[pl.ds(base + 512, RPT - 512)])
    pltpu.sync_copy(rowbuf.at[0, pl.ds(0, RPT - 512)],
                    den_sh.at[pl.ds(base + 512, RPT - 512)])

    pltpu.sync_copy(g_hbm, gv)
    g = gv[...]

    plsc.subcore_barrier()

    def _slot(buf, b):
        return buf.at[pl.ds(b * CHUNK, CHUNK)]

    def _fire_gathers(cc, b):
        pltpu.async_copy(asrc_hbm.at[srcix.at[cc]], _slot(abuf, b), sa[b])
        pltpu.async_copy(adst_hbm.at[dstix.at[cc]], _slot(bbuf, b), sb[b])
        pltpu.async_copy(h_hbm.at[srcix.at[cc]],
                         rowbuf.at[pl.ds(b * CHUNK, CHUNK)], sr[b])

    def _wait_alpha(cc, b):
        pltpu.make_async_copy(asrc_hbm.at[srcix.at[cc]], _slot(abuf, b),
                              sa[b]).wait()
        pltpu.make_async_copy(adst_hbm.at[dstix.at[cc]], _slot(bbuf, b),
                              sb[b]).wait()

    def _wait_rows(cc, b):
        pltpu.make_async_copy(h_hbm.at[srcix.at[cc]],
                              rowbuf.at[pl.ds(b * CHUNK, CHUNK)], sr[b]).wait()

    def _fire_den(cc, b):
        pltpu.async_copy(_slot(exbuf, b), den_sh.at[dstix.at[cc]], sd[b],
                         add=True)

    def _wait_den(cc, b):
        pltpu.make_async_copy(_slot(exbuf, b), den_sh.at[dstix.at[cc]],
                              sd[b]).wait()

    def _fire_rs(cc, b):
        pltpu.async_copy(rowbuf.at[pl.ds(b * CHUNK, CHUNK)],
                         acc_sh.at[dstix.at[cc]], ss[b], add=True)

    def _wait_rs(cc, b):
        pltpu.make_async_copy(rowbuf.at[pl.ds(b * CHUNK, CHUNK)],
                              acc_sh.at[dstix.at[cc]], ss[b]).wait()

    def _compute_ex(b):
        for k in range(CHUNK // 16):
            off = b * CHUNK + k * 16
            t = abuf[pl.ds(off, 16)] + bbuf[pl.ds(off, 16)]
            al = jnp.where(t >= 0, t, t * 0.2) - g
            exbuf[pl.ds(off, 16)] = jnp.exp(al)

    def _scale(b):
        def _edge(e2, carry):
            for u in range(2):
                r = b * CHUNK + e2 * 2 + u
                cf = plsc.load_gather(exbuf, [jnp.full((16,), r, jnp.int32)])
                for q in range(8):
                    rowbuf[r, pl.ds(q * 16, 16)] = (
                        rowbuf[r, pl.ds(q * 16, 16)] * cf)
            return carry
        lax.fori_loop(0, CHUNK // 2, _edge, 0)

    def _step(cc, b, kind):
        if kind >= 2:
            _wait_den(cc - 2, b)
        _wait_alpha(cc, b)
        _compute_ex(b)
        _fire_den(cc, b)
        if kind >= 1:
            _wait_rs(cc - 1, 1 - b)
        if kind <= 2:
            _fire_gathers(cc + 1, 1 - b)
        _wait_rows(cc, b)
        _scale(b)
        _fire_rs(cc, b)

    for hf in range(2):
        pltpu.sync_copy(src_hbm.at[w, hf], srcix)
        pltpu.sync_copy(dst_hbm.at[w, hf], dstix)

        _fire_gathers(0, 0)
        _step(0, 0, 0)
        _step(1, 1, 1)

        def _pass_body(t, carry):
            c0 = 2 * t
            _step(c0, 0, 2)
            _step(c0 + 1, 1, 2)
            return carry
        lax.fori_loop(1, (NCHP - 1) // 2, _pass_body, 0)

        _step(NCHP - 1, 0, 3)
        _wait_den(NCHP - 2, 1)
        _wait_den(NCHP - 1, 0)
        _wait_rs(NCHP - 1, 0)

    plsc.subcore_barrier()

    obase = cid * N_PAD + base
    for r in range(4):
        pltpu.sync_copy(acc_sh.at[pl.ds(base + r * 128, 128)], rowbuf)
        pltpu.sync_copy(rowbuf, acc_out.at[pl.ds(obase + r * 128, 128)])
        pltpu.sync_copy(den_sh.at[pl.ds(base + r * 128, 128)], abuf)
        pltpu.sync_copy(abuf, den_out.at[pl.ds(obase + r * 128, 128)])
    pltpu.sync_copy(acc_sh.at[pl.ds(base + 512, RPT - 512)],
                    rowbuf.at[pl.ds(0, RPT - 512)])
    pltpu.sync_copy(rowbuf.at[pl.ds(0, RPT - 512)],
                    acc_out.at[pl.ds(obase + 512, RPT - 512)])
    pltpu.sync_copy(den_sh.at[pl.ds(base + 512, RPT - 512)],
                    abuf.at[pl.ds(0, RPT - 512)])
    pltpu.sync_copy(abuf.at[pl.ds(0, RPT - 512)],
                    den_out.at[pl.ds(obase + 512, RPT - 512)])


@functools.lru_cache(maxsize=1)
def _sc_gat():
    return pl.kernel(
        _sc_gat_body,
        out_type=(
            jax.ShapeDtypeStruct((2 * N_PAD, D), jnp.float32),
            jax.ShapeDtypeStruct((2 * N_PAD,), jnp.float32),
        ),
        mesh=plsc.VectorSubcoreMesh(core_axis_name="c", subcore_axis_name="s",
                                    num_cores=2, num_subcores=16),
        scratch_types=[
            pltpu.VMEM((NCHP, CHUNK), jnp.int32),
            pltpu.VMEM((NCHP, CHUNK), jnp.int32),
            pltpu.VMEM((2 * CHUNK,), jnp.float32),
            pltpu.VMEM((2 * CHUNK,), jnp.float32),
            pltpu.VMEM((2 * CHUNK,), jnp.float32),
            pltpu.VMEM((2 * CHUNK, D), jnp.float32),
            pltpu.VMEM((16,), jnp.float32),
            pltpu.VMEM_SHARED((N_PAD, D), jnp.float32),
            pltpu.VMEM_SHARED((N_PAD,), jnp.float32),
            pltpu.SemaphoreType.DMA,
            pltpu.SemaphoreType.DMA,
            pltpu.SemaphoreType.DMA,
            pltpu.SemaphoreType.DMA,
            pltpu.SemaphoreType.DMA,
            pltpu.SemaphoreType.DMA,
            pltpu.SemaphoreType.DMA,
            pltpu.SemaphoreType.DMA,
            pltpu.SemaphoreType.DMA,
            pltpu.SemaphoreType.DMA,
        ],
        compiler_params=pltpu.CompilerParams(needs_layout_passes=False),
    )


def kernel(x, edge_index, W1, b1, Wg1, as1, ad1, bg1, Wg2, as2, ad2, bg2, W2, b2):
    src = edge_index[0].reshape(NW, EPT)
    dst = edge_index[1].reshape(NW, EPT)
    src = jnp.pad(src, ((0, 0), (0, EPT_PAD - EPT))).reshape(NW, 2, NCHP, CHUNK)
    dst = jnp.pad(dst, ((0, 0), (0, EPT_PAD - EPT)),
                  constant_values=N).reshape(NW, 2, NCHP, CHUNK)

    xp = jnp.pad(x, ((0, N_PAD - N), (0, 0)))
    r1 = lambda v: v.reshape(1, -1)

    h1, asrc1, adst1, g1 = _tc_proj1(xp, W1, r1(b1), Wg1, r1(as1), r1(ad1))
    acc1, den1 = _sc_gat()(h1, src, dst, asrc1[:N, 0], adst1[:N, 0],
                           g1.reshape(16))
    acc1 = acc1.reshape(2, N_PAD, D)
    den1 = den1.reshape(2, N_PAD)

    h2, asrc2, adst2, g2 = _tc_proj2(acc1, den1[..., None], r1(bg1), Wg2,
                                     r1(as2), r1(ad2))
    acc2, den2 = _sc_gat()(h2, src, dst, asrc2[:N, 0], adst2[:N, 0],
                           g2.reshape(16))
    acc2 = acc2.reshape(2, N_PAD, D)
    den2 = den2.reshape(2, N_PAD)

    out = _tc_final(acc2, den2[..., None], r1(bg2), W2, b2.reshape(1, C))
    return out[:N]


# speedup vs baseline: 26.8929x; 1.0021x over previous
import functools

import jax
import jax.numpy as jnp
from jax import lax
from jax.experimental import pallas as pl
from jax.experimental.pallas import tpu as pltpu
from jax.experimental.pallas import tpu_sc as plsc

N = 10000
D = 128
C = 40
E = 320000

NW = 32
EPT = E // NW
CHUNK = 64
NCH = 2 * ((EPT + 2 * CHUNK - 1) // (2 * CHUNK))
EPT_PAD = NCH * CHUNK
NCHP = NCH // 2
N_PAD = 10112
RPT = N_PAD // 16
BLK = 632


def _proj_tail(h, as_ref, ad_ref, asrc_ref, adst_ref, g_ref, mx_ref):
    i = pl.program_id(0)
    asrc = jnp.sum(h * as_ref[...], axis=1, keepdims=True)
    adst = jnp.sum(h * ad_ref[...], axis=1, keepdims=True)
    asrc_ref[...] = asrc
    adst_ref[...] = adst

    @pl.when(i == 0)
    def _():
        mx_ref[0] = -jnp.inf
        mx_ref[1] = -jnp.inf

    mx_ref[0] = jnp.maximum(mx_ref[0], jnp.max(asrc))
    mx_ref[1] = jnp.maximum(mx_ref[1], jnp.max(adst))

    @pl.when(i == pl.num_programs(0) - 1)
    def _():
        s = mx_ref[0] + mx_ref[1]
        g = jnp.where(s >= 0, s, 0.2 * s)
        g_ref[...] = jnp.full((1, 16), g, jnp.float32)


def _tc1_body(x_ref, w1_ref, b1_ref, wg_ref, as_ref, ad_ref,
              h_ref, asrc_ref, adst_ref, g_ref, mx_ref):
    h0 = jnp.maximum(jnp.dot(x_ref[...], w1_ref[...],
                             preferred_element_type=jnp.float32) + b1_ref[...], 0.0)
    h = jnp.dot(h0, wg_ref[...], preferred_element_type=jnp.float32)
    h_ref[...] = h
    _proj_tail(h, as_ref, ad_ref, asrc_ref, adst_ref, g_ref, mx_ref)


def _tc2_body(acc_ref, den_ref, bg_ref, wg_ref, as_ref, ad_ref,
              h_ref, asrc_ref, adst_ref, g_ref, mx_ref):
    agg = acc_ref[0] + acc_ref[1]
    dn = den_ref[0] + den_ref[1]
    out = agg / (dn + 1e-16) + bg_ref[...]
    h1 = jnp.maximum(out, 0.0)
    h = jnp.dot(h1, wg_ref[...], preferred_element_type=jnp.float32)
    h_ref[...] = h
    _proj_tail(h, as_ref, ad_ref, asrc_ref, adst_ref, g_ref, mx_ref)


def _tc3_body(acc_ref, den_ref, bg_ref, w2_ref, b2_ref, o_ref):
    agg = acc_ref[0] + acc_ref[1]
    dn = den_ref[0] + den_ref[1]
    out = agg / (dn + 1e-16) + bg_ref[...]
    logits = jnp.dot(out, w2_ref[...],
                     preferred_element_type=jnp.float32) + b2_ref[...]
    m = jnp.max(logits, axis=1, keepdims=True)
    ls = logits - m
    o_ref[...] = ls - jnp.log(jnp.sum(jnp.exp(ls), axis=1, keepdims=True))


def _tc_proj1(x, w1, b1, wg, a_s, a_d):
    n = x.shape[0]
    return pl.pallas_call(
        _tc1_body,
        grid=(n // BLK,),
        in_specs=[
            pl.BlockSpec((BLK, D), lambda i: (i, 0)),
            pl.BlockSpec((D, D), lambda i: (0, 0)),
            pl.BlockSpec((1, D), lambda i: (0, 0)),
            pl.BlockSpec((D, D), lambda i: (0, 0)),
            pl.BlockSpec((1, D), lambda i: (0, 0)),
            pl.BlockSpec((1, D), lambda i: (0, 0)),
        ],
        out_specs=[
            pl.BlockSpec((BLK, D), lambda i: (i, 0)),
            pl.BlockSpec((BLK, 1), lambda i: (i, 0)),
            pl.BlockSpec((BLK, 1), lambda i: (i, 0)),
            pl.BlockSpec((1, 16), lambda i: (0, 0)),
        ],
        out_shape=[
            jax.ShapeDtypeStruct((n, D), jnp.float32),
            jax.ShapeDtypeStruct((n, 1), jnp.float32),
            jax.ShapeDtypeStruct((n, 1), jnp.float32),
            jax.ShapeDtypeStruct((1, 16), jnp.float32),
        ],
        scratch_shapes=[pltpu.SMEM((2,), jnp.float32)],
    )(x, w1, b1, wg, a_s, a_d)


def _tc_proj2(acc, den, bg, wg, a_s, a_d):
    n = acc.shape[1]
    return pl.pallas_call(
        _tc2_body,
        grid=(n // BLK,),
        in_specs=[
            pl.BlockSpec((2, BLK, D), lambda i: (0, i, 0)),
            pl.BlockSpec((2, BLK, 1), lambda i: (0, i, 0)),
            pl.BlockSpec((1, D), lambda i: (0, 0)),
            pl.BlockSpec((D, D), lambda i: (0, 0)),
            pl.BlockSpec((1, D), lambda i: (0, 0)),
            pl.BlockSpec((1, D), lambda i: (0, 0)),
        ],
        out_specs=[
            pl.BlockSpec((BLK, D), lambda i: (i, 0)),
            pl.BlockSpec((BLK, 1), lambda i: (i, 0)),
            pl.BlockSpec((BLK, 1), lambda i: (i, 0)),
            pl.BlockSpec((1, 16), lambda i: (0, 0)),
        ],
        out_shape=[
            jax.ShapeDtypeStruct((n, D), jnp.float32),
            jax.ShapeDtypeStruct((n, 1), jnp.float32),
            jax.ShapeDtypeStruct((n, 1), jnp.float32),
            jax.ShapeDtypeStruct((1, 16), jnp.float32),
        ],
        scratch_shapes=[pltpu.SMEM((2,), jnp.float32)],
    )(acc, den, bg, wg, a_s, a_d)


def _tc_final(acc, den, bg, w2, b2):
    n = acc.shape[1]
    return pl.pallas_call(
        _tc3_body,
        grid=(n // BLK,),
        in_specs=[
            pl.BlockSpec((2, BLK, D), lambda i: (0, i, 0)),
            pl.BlockSpec((2, BLK, 1), lambda i: (0, i, 0)),
            pl.BlockSpec((1, D), lambda i: (0, 0)),
            pl.BlockSpec((D, C), lambda i: (0, 0)),
            pl.BlockSpec((1, C), lambda i: (0, 0)),
        ],
        out_specs=pl.BlockSpec((BLK, C), lambda i: (i, 0)),
        out_shape=jax.ShapeDtypeStruct((n, C), jnp.float32),
    )(acc, den, bg, w2, b2)


def _sc_gat_body(h_hbm, src_hbm, dst_hbm, asrc_hbm, adst_hbm, g_hbm,
                 acc_out, den_out,
                 srcix, dstix, abuf, bbuf, exbuf, rowbuf, gv,
                 acc_sh, den_sh,
                 sem_a0, sem_a1, sem_b0, sem_b1, sem_r0, sem_r1,
                 sem_d0, sem_d1, sem_s0, sem_s1):
    cid = lax.axis_index("c")
    s = lax.axis_index("s")
    w = cid * 16 + s
    base = s * RPT
    sa = (sem_a0, sem_a1)
    sb = (sem_b0, sem_b1)
    sr = (sem_r0, sem_r1)
    sd = (sem_d0, sem_d1)
    ss = (sem_s0, sem_s1)

    def _zrow(i, carry):
        for q in range(8):
            rowbuf[i, pl.ds(q * 16, 16)] = jnp.zeros((16,), jnp.float32)
        return carry
    lax.fori_loop(0, 2 * CHUNK, _zrow, 0)

    zdescs = []
    for r in range(4):
        zdescs.append(pltpu.async_copy(
            rowbuf, acc_sh.at[pl.ds(base + r * 128, 128)], sem_s0))
        zdescs.append(pltpu.async_copy(
            rowbuf.at[0], den_sh.at[pl.ds(base + r * 128, 128)], sem_s1))
    zdescs.append(pltpu.async_copy(
        rowbuf.at[pl.ds(0, RPT - 512)],
        acc_sh.at[pl.ds(base + 512, RPT - 512)], sem_s0))
    zdescs.append(pltpu.async_copy(
        rowbuf.at[0, pl.ds(0, RPT - 512)],
        den_sh.at[pl.ds(base + 512, RPT - 512)], sem_s1))
    for dsc in zdescs:
        dsc.wait()

    pltpu.sync_copy(g_hbm, gv)
    g = gv[...]

    plsc.subcore_barrier()

    def _slot(buf, b):
        return buf.at[pl.ds(b * CHUNK, CHUNK)]

    def _fire_gathers(cc, b):
        pltpu.async_copy(asrc_hbm.at[srcix.at[cc]], _slot(abuf, b), sa[b])
        pltpu.async_copy(adst_hbm.at[dstix.at[cc]], _slot(bbuf, b), sb[b])
        pltpu.async_copy(h_hbm.at[srcix.at[cc]],
                         rowbuf.at[pl.ds(b * CHUNK, CHUNK)], sr[b])

    def _wait_alpha(cc, b):
        pltpu.make_async_copy(asrc_hbm.at[srcix.at[cc]], _slot(abuf, b),
                              sa[b]).wait()
        pltpu.make_async_copy(adst_hbm.at[dstix.at[cc]], _slot(bbuf, b),
                              sb[b]).wait()

    def _wait_rows(cc, b):
        pltpu.make_async_copy(h_hbm.at[srcix.at[cc]],
                              rowbuf.at[pl.ds(b * CHUNK, CHUNK)], sr[b]).wait()

    def _fire_den(cc, b):
        pltpu.async_copy(_slot(exbuf, b), den_sh.at[dstix.at[cc]], sd[b],
                         add=True)

    def _wait_den(cc, b):
        pltpu.make_async_copy(_slot(exbuf, b), den_sh.at[dstix.at[cc]],
                              sd[b]).wait()

    def _fire_rs(cc, b):
        pltpu.async_copy(rowbuf.at[pl.ds(b * CHUNK, CHUNK)],
                         acc_sh.at[dstix.at[cc]], ss[b], add=True)

    def _wait_rs(cc, b):
        pltpu.make_async_copy(rowbuf.at[pl.ds(b * CHUNK, CHUNK)],
                              acc_sh.at[dstix.at[cc]], ss[b]).wait()

    def _compute_ex(b):
        for k in range(CHUNK // 16):
            off = b * CHUNK + k * 16
            t = abuf[pl.ds(off, 16)] + bbuf[pl.ds(off, 16)]
            al = jnp.where(t >= 0, t, t * 0.2) - g
            exbuf[pl.ds(off, 16)] = jnp.exp(al)

    def _scale(b):
        def _edge(e4, carry):
            for u in range(4):
                r = b * CHUNK + e4 * 4 + u
                cf = plsc.load_gather(exbuf, [jnp.full((16,), r, jnp.int32)])
                for q in range(8):
                    rowbuf[r, pl.ds(q * 16, 16)] = (
                        rowbuf[r, pl.ds(q * 16, 16)] * cf)
            return carry
        lax.fori_loop(0, CHUNK // 4, _edge, 0)

    def _step(cc, b, kind):
        if kind >= 2:
            _wait_den(cc - 2, b)
        _wait_alpha(cc, b)
        _compute_ex(b)
        _fire_den(cc, b)
        if kind >= 1:
            _wait_rs(cc - 1, 1 - b)
        if kind <= 2:
            _fire_gathers(cc + 1, 1 - b)
        _wait_rows(cc, b)
        _scale(b)
        _fire_rs(cc, b)

    for hf in range(2):
        pltpu.sync_copy(src_hbm.at[w, hf], srcix)
        pltpu.sync_copy(dst_hbm.at[w, hf], dstix)

        _fire_gathers(0, 0)
        _step(0, 0, 0)
        _step(1, 1, 1)

        def _pass_body(t, carry):
            c0 = 2 * t
            _step(c0, 0, 2)
            _step(c0 + 1, 1, 2)
            return carry
        lax.fori_loop(1, (NCHP - 1) // 2, _pass_body, 0)

        _step(NCHP - 1, 0, 3)
        _wait_den(NCHP - 2, 1)
        _wait_den(NCHP - 1, 0)
        _wait_rs(NCHP - 1, 0)

    plsc.subcore_barrier()

    obase = cid * N_PAD + base
    for r in range(4):
        pltpu.sync_copy(acc_sh.at[pl.ds(base + r * 128, 128)], rowbuf)
        pltpu.sync_copy(rowbuf, acc_out.at[pl.ds(obase + r * 128, 128)])
        pltpu.sync_copy(den_sh.at[pl.ds(base + r * 128, 128)], abuf)
        pltpu.sync_copy(abuf, den_out.at[pl.ds(obase + r * 128, 128)])
    pltpu.sync_copy(acc_sh.at[pl.ds(base + 512, RPT - 512)],
                    rowbuf.at[pl.ds(0, RPT - 512)])
    pltpu.sync_copy(rowbuf.at[pl.ds(0, RPT - 512)],
                    acc_out.at[pl.ds(obase + 512, RPT - 512)])
    pltpu.sync_copy(den_sh.at[pl.ds(base + 512, RPT - 512)],
                    abuf.at[pl.ds(0, RPT - 512)])
    pltpu.sync_copy(abuf.at[pl.ds(0, RPT - 512)],
                    den_out.at[pl.ds(obase + 512, RPT - 512)])


@functools.lru_cache(maxsize=1)
def _sc_gat():
    return pl.kernel(
        _sc_gat_body,
        out_type=(
            jax.ShapeDtypeStruct((2 * N_PAD, D), jnp.float32),
            jax.ShapeDtypeStruct((2 * N_PAD,), jnp.float32),
        ),
        mesh=plsc.VectorSubcoreMesh(core_axis_name="c", subcore_axis_name="s",
                                    num_cores=2, num_subcores=16),
        scratch_types=[
            pltpu.VMEM((NCHP, CHUNK), jnp.int32),
            pltpu.VMEM((NCHP, CHUNK), jnp.int32),
            pltpu.VMEM((2 * CHUNK,), jnp.float32),
            pltpu.VMEM((2 * CHUNK,), jnp.float32),
            pltpu.VMEM((2 * CHUNK,), jnp.float32),
            pltpu.VMEM((2 * CHUNK, D), jnp.float32),
            pltpu.VMEM((16,), jnp.float32),
            pltpu.VMEM_SHARED((N_PAD, D), jnp.float32),
            pltpu.VMEM_SHARED((N_PAD,), jnp.float32),
            pltpu.SemaphoreType.DMA,
            pltpu.SemaphoreType.DMA,
            pltpu.SemaphoreType.DMA,
            pltpu.SemaphoreType.DMA,
            pltpu.SemaphoreType.DMA,
            pltpu.SemaphoreType.DMA,
            pltpu.SemaphoreType.DMA,
            pltpu.SemaphoreType.DMA,
            pltpu.SemaphoreType.DMA,
            pltpu.SemaphoreType.DMA,
        ],
        compiler_params=pltpu.CompilerParams(needs_layout_passes=False),
    )


def kernel(x, edge_index, W1, b1, Wg1, as1, ad1, bg1, Wg2, as2, ad2, bg2, W2, b2):
    src = edge_index[0].reshape(NW, EPT)
    dst = edge_index[1].reshape(NW, EPT)
    src = jnp.pad(src, ((0, 0), (0, EPT_PAD - EPT))).reshape(NW, 2, NCHP, CHUNK)
    dst = jnp.pad(dst, ((0, 0), (0, EPT_PAD - EPT)),
                  constant_values=N).reshape(NW, 2, NCHP, CHUNK)

    xp = jnp.pad(x, ((0, N_PAD - N), (0, 0)))
    r1 = lambda v: v.reshape(1, -1)

    h1, asrc1, adst1, g1 = _tc_proj1(xp, W1, r1(b1), Wg1, r1(as1), r1(ad1))
    acc1, den1 = _sc_gat()(h1, src, dst, asrc1[:N, 0], adst1[:N, 0],
                           g1.reshape(16))
    acc1 = acc1.reshape(2, N_PAD, D)
    den1 = den1.reshape(2, N_PAD)

    h2, asrc2, adst2, g2 = _tc_proj2(acc1, den1[..., None], r1(bg1), Wg2,
                                     r1(as2), r1(ad2))
    acc2, den2 = _sc_gat()(h2, src, dst, asrc2[:N, 0], adst2[:N, 0],
                           g2.reshape(16))
    acc2 = acc2.reshape(2, N_PAD, D)
    den2 = den2.reshape(2, N_PAD)

    out = _tc_final(acc2, den2[..., None], r1(bg2), W2, b2.reshape(1, C))
    return out[:N]
